# baseline (speedup 1.0000x reference)
import jax
import jax.numpy as jnp
from jax.experimental import pallas as pl
from jax.experimental.pallas import tpu as pltpu

_N = 3136
_NP = 3200
_K = 9
_TQ = 640
_NQT = _NP // _TQ

_BIG = 3e38


def _knn_mlp_kernel(xk_ref, xq_ref, eps_ref, w1_ref, b1_ref, w2_ref, b2_ref,
                    out_ref, st_ref):
    X = xk_ref[0]
    Q = xq_ref[0]
    C = X.shape[0]

    sqk = jnp.sum(X * X, axis=0, keepdims=True)
    kiota1 = jax.lax.broadcasted_iota(jnp.int32, (1, _NP), 1)
    sqk = jnp.where(kiota1 >= _N, _BIG, sqk)
    Xb = X.astype(jnp.bfloat16)
    gram = jax.lax.dot_general(
        Q.astype(jnp.bfloat16), Xb,
        (((0,), (0,)), ((), ())),
        preferred_element_type=jnp.float32)
    score = sqk - 2.0 * gram

    t9s = []
    for g in range(_TQ // 64):
        rows = slice(g * 64, (g + 1) * 64)
        r1 = jnp.full((64, 128), _BIG, jnp.float32)
        r2 = r1
        r3 = r1
        r4 = r1
        for c in range(_NP // 128):
            v = score[rows, c * 128:(c + 1) * 128]
            t = jnp.maximum(r1, v)
            r1 = jnp.minimum(r1, v)
            t2 = jnp.maximum(r2, t)
            r2 = jnp.minimum(r2, t)
            t3 = jnp.maximum(r3, t2)
            r3 = jnp.minimum(r3, t2)
            r4 = jnp.minimum(r4, t3)
        tg = jnp.min(r1, axis=1, keepdims=True)
        for _ in range(_K - 1):
            hit = r1 == tg
            r1 = jnp.where(hit, r2, r1)
            r2 = jnp.where(hit, r3, r2)
            r3 = jnp.where(hit, r4, r3)
            r4 = jnp.where(hit, _BIG, r4)
            tg = jnp.min(r1, axis=1, keepdims=True)
        t9s.append(tg)
    t9 = jnp.concatenate(t9s, axis=0)
    mask = (score <= t9).astype(jnp.bfloat16)

    nsum = jax.lax.dot_general(
        Xb, mask, (((1,), (1,)), ((), ())),
        preferred_element_type=jnp.float32)

    h = (1.0 + eps_ref[...]) * Q + nsum
    a1 = jax.lax.dot_general(
        w1_ref[...].astype(jnp.bfloat16), h.astype(jnp.bfloat16),
        (((1,), (0,)), ((), ())),
        preferred_element_type=jnp.float32) + b1_ref[...]
    a1 = jnp.maximum(a1, 0.0)
    o = jax.lax.dot_general(
        w2_ref[...].astype(jnp.bfloat16), a1.astype(jnp.bfloat16),
        (((1,), (0,)), ((), ())),
        preferred_element_type=jnp.float32) + b2_ref[...]

    qcol = pl.program_id(1) * _TQ + jax.lax.broadcasted_iota(
        jnp.int32, (C, _TQ), 1)
    o = jnp.where(qcol < _N, o, 0.0)
    out_ref[0] = o

    psum = jnp.sum(o, axis=1, keepdims=True)
    psq = jnp.sum(o * o, axis=1, keepdims=True)
    st_ref[0, 0] = jnp.concatenate(
        [psum, psq, jnp.zeros((C, 6), jnp.float32)], axis=1)


def _bn_res_kernel(o_ref, x_ref, sc_ref, sh_ref, y_ref):
    o = o_ref[0, :, :_N]
    r = x_ref[0]
    y_ref[0] = jnp.maximum(o * sc_ref[...] + sh_ref[...] + r, 0.0)


def kernel(x, eps, W1, b1, W2, b2, gamma, beta):
    B, C, H, W = x.shape
    n = H * W
    xr = x.reshape(B, C, n)
    xp = jnp.pad(xr, ((0, 0), (0, 0), (0, _NP - n)))
    eps2 = jnp.reshape(eps, (1, 1)).astype(jnp.float32)
    b1c = b1.reshape(C, 1)
    b2c = b2.reshape(C, 1)

    out, stats = pl.pallas_call(
        _knn_mlp_kernel,
        grid=(B, _NQT),
        in_specs=[
            pl.BlockSpec((1, C, _NP), lambda b, q: (b, 0, 0)),
            pl.BlockSpec((1, C, _TQ), lambda b, q: (b, 0, q)),
            pl.BlockSpec((1, 1), lambda b, q: (0, 0)),
            pl.BlockSpec((C, C), lambda b, q: (0, 0)),
            pl.BlockSpec((C, 1), lambda b, q: (0, 0)),
            pl.BlockSpec((C, C), lambda b, q: (0, 0)),
            pl.BlockSpec((C, 1), lambda b, q: (0, 0)),
        ],
        out_specs=[
            pl.BlockSpec((1, C, _TQ), lambda b, q: (b, 0, q)),
            pl.BlockSpec((1, 1, C, 8), lambda b, q: (b, q, 0, 0)),
        ],
        out_shape=[
            jax.ShapeDtypeStruct((B, C, _NP), jnp.float32),
            jax.ShapeDtypeStruct((B, _NQT, C, 8), jnp.float32),
        ],
        compiler_params=pltpu.CompilerParams(
            dimension_semantics=("parallel", "arbitrary")),
    )(xp, xp, eps2, W1, b1c, W2, b2c)

    s = jnp.sum(stats, axis=(0, 1))
    cnt = jnp.float32(B * n)
    mean = s[:, 0] / cnt
    var = s[:, 1] / cnt - mean * mean
    inv = jax.lax.rsqrt(var + 1e-5)
    scale = (gamma * inv).reshape(C, 1)
    shift = (beta - mean * gamma * inv).reshape(C, 1)

    y = pl.pallas_call(
        _bn_res_kernel,
        grid=(B,),
        in_specs=[
            pl.BlockSpec((1, C, _NP), lambda b: (b, 0, 0)),
            pl.BlockSpec((1, C, n), lambda b: (b, 0, 0)),
            pl.BlockSpec((C, 1), lambda b: (0, 0)),
            pl.BlockSpec((C, 1), lambda b: (0, 0)),
        ],
        out_specs=pl.BlockSpec((1, C, n), lambda b: (b, 0, 0)),
        out_shape=jax.ShapeDtypeStruct((B, C, n), jnp.float32),
        compiler_params=pltpu.CompilerParams(
            dimension_semantics=("parallel",)),
    )(out, xr, scale, shift)

    return y.reshape(B, C, H, W)


# speedup vs baseline: 1.0282x; 1.0282x over previous
import jax
import jax.numpy as jnp
from jax.experimental import pallas as pl
from jax.experimental.pallas import tpu as pltpu

_N = 3136
_NP = 3200
_K = 9
_TQ = 640
_NQT = _NP // _TQ

_BIG = 3e38


def _knn_mlp_kernel(xk_ref, xq_ref, eps_ref, w1_ref, b1_ref, w2_ref, b2_ref,
                    out_ref, st_ref):
    X = xk_ref[0]
    Q = xq_ref[0]
    C = X.shape[0]

    sqk = jnp.sum(X * X, axis=0, keepdims=True)
    kiota1 = jax.lax.broadcasted_iota(jnp.int32, (1, _NP), 1)
    sqk = jnp.where(kiota1 >= _N, _BIG, sqk)
    Xb = X.astype(jnp.bfloat16)
    gram = jax.lax.dot_general(
        Q.astype(jnp.bfloat16), Xb,
        (((0,), (0,)), ((), ())),
        preferred_element_type=jnp.float32)
    score = sqk - 2.0 * gram

    r1 = jnp.full((_TQ, 128), _BIG, jnp.float32)
    r2 = r1
    r3 = r1
    r4 = r1
    for c in range(_NP // 128):
        v = score[:, c * 128:(c + 1) * 128]
        t = jnp.maximum(r1, v)
        r1 = jnp.minimum(r1, v)
        t2 = jnp.maximum(r2, t)
        r2 = jnp.minimum(r2, t)
        t3 = jnp.maximum(r3, t2)
        r3 = jnp.minimum(r3, t2)
        r4 = jnp.minimum(r4, t3)
    t9 = jnp.min(r1, axis=1, keepdims=True)
    for _ in range(_K - 1):
        hit = r1 == t9
        r1 = jnp.where(hit, r2, r1)
        r2 = jnp.where(hit, r3, r2)
        r3 = jnp.where(hit, r4, r3)
        r4 = jnp.where(hit, _BIG, r4)
        t9 = jnp.min(r1, axis=1, keepdims=True)
    mask = (score <= t9).astype(jnp.bfloat16)

    nsum = jax.lax.dot_general(
        Xb, mask, (((1,), (1,)), ((), ())),
        preferred_element_type=jnp.float32)

    h = (1.0 + eps_ref[...]) * Q + nsum
    a1 = jax.lax.dot_general(
        w1_ref[...].astype(jnp.bfloat16), h.astype(jnp.bfloat16),
        (((1,), (0,)), ((), ())),
        preferred_element_type=jnp.float32) + b1_ref[...]
    a1 = jnp.maximum(a1, 0.0)
    o = jax.lax.dot_general(
        w2_ref[...].astype(jnp.bfloat16), a1.astype(jnp.bfloat16),
        (((1,), (0,)), ((), ())),
        preferred_element_type=jnp.float32) + b2_ref[...]

    qcol = pl.program_id(1) * _TQ + jax.lax.broadcasted_iota(
        jnp.int32, (C, _TQ), 1)
    o = jnp.where(qcol < _N, o, 0.0)
    out_ref[0] = o

    psum = jnp.sum(o, axis=1, keepdims=True)
    psq = jnp.sum(o * o, axis=1, keepdims=True)
    st_ref[0, 0] = jnp.concatenate(
        [psum, psq, jnp.zeros((C, 6), jnp.float32)], axis=1)


def _bn_res_kernel(o_ref, x_ref, sc_ref, sh_ref, y_ref):
    o = o_ref[0, :, :_N]
    r = x_ref[0]
    y_ref[0] = jnp.maximum(o * sc_ref[...] + sh_ref[...] + r, 0.0)


def kernel(x, eps, W1, b1, W2, b2, gamma, beta):
    B, C, H, W = x.shape
    n = H * W
    xr = x.reshape(B, C, n)
    xp = jnp.pad(xr, ((0, 0), (0, 0), (0, _NP - n)))
    eps2 = jnp.reshape(eps, (1, 1)).astype(jnp.float32)
    b1c = b1.reshape(C, 1)
    b2c = b2.reshape(C, 1)

    out, stats = pl.pallas_call(
        _knn_mlp_kernel,
        grid=(B, _NQT),
        in_specs=[
            pl.BlockSpec((1, C, _NP), lambda b, q: (b, 0, 0)),
            pl.BlockSpec((1, C, _TQ), lambda b, q: (b, 0, q)),
            pl.BlockSpec((1, 1), lambda b, q: (0, 0)),
            pl.BlockSpec((C, C), lambda b, q: (0, 0)),
            pl.BlockSpec((C, 1), lambda b, q: (0, 0)),
            pl.BlockSpec((C, C), lambda b, q: (0, 0)),
            pl.BlockSpec((C, 1), lambda b, q: (0, 0)),
        ],
        out_specs=[
            pl.BlockSpec((1, C, _TQ), lambda b, q: (b, 0, q)),
            pl.BlockSpec((1, 1, C, 8), lambda b, q: (b, q, 0, 0)),
        ],
        out_shape=[
            jax.ShapeDtypeStruct((B, C, _NP), jnp.float32),
            jax.ShapeDtypeStruct((B, _NQT, C, 8), jnp.float32),
        ],
        compiler_params=pltpu.CompilerParams(
            dimension_semantics=("parallel", "arbitrary")),
    )(xp, xp, eps2, W1, b1c, W2, b2c)

    s = jnp.sum(stats, axis=(0, 1))
    cnt = jnp.float32(B * n)
    mean = s[:, 0] / cnt
    var = s[:, 1] / cnt - mean * mean
    inv = jax.lax.rsqrt(var + 1e-5)
    scale = (gamma * inv).reshape(C, 1)
    shift = (beta - mean * gamma * inv).reshape(C, 1)

    y = pl.pallas_call(
        _bn_res_kernel,
        grid=(B,),
        in_specs=[
            pl.BlockSpec((1, C, _NP), lambda b: (b, 0, 0)),
            pl.BlockSpec((1, C, n), lambda b: (b, 0, 0)),
            pl.BlockSpec((C, 1), lambda b: (0, 0)),
            pl.BlockSpec((C, 1), lambda b: (0, 0)),
        ],
        out_specs=pl.BlockSpec((1, C, n), lambda b: (b, 0, 0)),
        out_shape=jax.ShapeDtypeStruct((B, C, n), jnp.float32),
        compiler_params=pltpu.CompilerParams(
            dimension_semantics=("parallel",)),
    )(out, xr, scale, shift)

    return y.reshape(B, C, H, W)
